# baseline (speedup 1.0000x reference)
import functools

import jax
import jax.numpy as jnp
from jax import lax
from jax.experimental import pallas as pl
from jax.experimental.pallas import tpu as pltpu
from jax.experimental.pallas import tpu_sc as plsc

G = 16
K = 1024
D = 64
TILE_B = 512


def _search_kernel(z_ref, cb_ref, idx_ref, flat_ref, com_ref, cod_ref,
                   ent_ref, perp_ref, csq_ref, loss_ref, hist_ref, *,
                   batch):
    i = pl.program_id(0)
    n_steps = pl.num_programs(0)

    @pl.when(i == 0)
    def _init():
        loss_ref[:, :] = jnp.zeros((1, 1), jnp.float32)
        hist_ref[:, :] = jnp.zeros((1, K), jnp.float32)
        for g in range(G):
            cb = cb_ref[g]
            csq_ref[g, :, :] = jnp.sum(cb * cb, axis=1)[None, :]

    z_all = z_ref[:, :]
    zneg = z_all * -2.0
    loss_part = jnp.zeros((1, 1), jnp.float32)
    hist_part = jnp.zeros((1, K), jnp.float32)
    ones_row = jnp.ones((1, TILE_B), jnp.float32)
    idx_cols = []
    flat_cols = []
    for g in range(G):
        zg = z_all[:, g * D:(g + 1) * D]
        cross2 = lax.dot_general(
            zneg[:, g * D:(g + 1) * D], cb_ref[g],
            (((1,), (1,)), ((), ())),
            preferred_element_type=jnp.float32)
        z_sq = jnp.sum(zg * zg, axis=1, keepdims=True)
        dist = (z_sq + cross2) + csq_ref[g]
        mind = jnp.min(dist, axis=1, keepdims=True)
        idx = jnp.argmin(dist, axis=1).astype(jnp.int32)
        maskf = jnp.where(dist <= mind, 1.0, 0.0).astype(jnp.float32)
        hist_part = hist_part + jnp.dot(
            ones_row, maskf, preferred_element_type=jnp.float32)
        loss_part = loss_part + jnp.sum(mind).reshape(1, 1)
        idx_cols.append(idx[:, None])
        flat_cols.append(idx[:, None] + g * K)

    idx_ref[:, :] = jnp.concatenate(idx_cols, axis=1)
    flat_ref[:, :] = jnp.concatenate(flat_cols, axis=1)
    hist_ref[:, :] += hist_part
    loss_ref[:, :] += loss_part

    @pl.when(i == n_steps - 1)
    def _finalize():
        loss = loss_ref[:, :] * (1.0 / jnp.float32(batch * G * D))
        com_ref[:, :] = loss
        cod_ref[:, :] = loss
        usage = hist_ref[:, :] * (1.0 / jnp.float32(batch * G))
        ent = -jnp.sum(usage * jnp.log(usage + 1e-10)).reshape(1, 1)
        ent_ref[:, :] = ent
        perp_ref[:, :] = jnp.exp(ent)


def _search(z, codebook):
    batch = z.shape[0]
    n_tiles = batch // TILE_B
    out_shapes = (
        jax.ShapeDtypeStruct((batch, G), jnp.int32),
        jax.ShapeDtypeStruct((batch, G), jnp.int32),
        jax.ShapeDtypeStruct((1, 1), jnp.float32),
        jax.ShapeDtypeStruct((1, 1), jnp.float32),
        jax.ShapeDtypeStruct((1, 1), jnp.float32),
        jax.ShapeDtypeStruct((1, 1), jnp.float32),
    )
    in_specs = [
        pl.BlockSpec((TILE_B, G * D), lambda i: (i, 0)),
        pl.BlockSpec((G, K, D), lambda i: (0, 0, 0)),
    ]
    out_specs = (
        pl.BlockSpec((TILE_B, G), lambda i: (i, 0)),
        pl.BlockSpec((TILE_B, G), lambda i: (i, 0)),
        pl.BlockSpec((1, 1), lambda i: (0, 0)),
        pl.BlockSpec((1, 1), lambda i: (0, 0)),
        pl.BlockSpec((1, 1), lambda i: (0, 0)),
        pl.BlockSpec((1, 1), lambda i: (0, 0)),
    )
    scratch_shapes = [
        pltpu.VMEM((G, 1, K), jnp.float32),
        pltpu.VMEM((1, 1), jnp.float32),
        pltpu.VMEM((1, K), jnp.float32),
    ]
    return pl.pallas_call(
        functools.partial(_search_kernel, batch=batch),
        grid=(n_tiles,),
        in_specs=in_specs,
        out_specs=out_specs,
        out_shape=out_shapes,
        scratch_shapes=scratch_shapes,
    )(z, codebook)


def _make_sc_gather(n_rows, chunk):
    info = plsc.get_sparse_core_info()
    nw = info.num_cores * info.num_subcores
    rows_per_w = n_rows // nw
    n_chunks = rows_per_w // chunk
    mesh = plsc.VectorSubcoreMesh(core_axis_name="c", subcore_axis_name="s")

    @functools.partial(
        pl.kernel, mesh=mesh,
        compiler_params=pltpu.CompilerParams(use_tc_tiling_on_sc=False),
        out_type=jax.ShapeDtypeStruct((n_rows, D), jnp.float32),
        scratch_types=[
            pltpu.VMEM((chunk,), jnp.int32),
            pltpu.VMEM((chunk, D), jnp.float32),
            pltpu.SemaphoreType.DMA,
        ],
    )
    def gather_k(table_hbm, flat_hbm, out_hbm, idx_v, rows_v, sem):
        wid = lax.axis_index("s") * info.num_cores + lax.axis_index("c")
        base = wid * rows_per_w
        for c in range(n_chunks):
            off = base + c * chunk
            pltpu.sync_copy(flat_hbm.at[pl.ds(off, chunk)], idx_v)
            pltpu.async_copy(table_hbm.at[idx_v], rows_v, sem).wait()
            pltpu.sync_copy(rows_v, out_hbm.at[pl.ds(off, chunk)])

    return gather_k


def kernel(z, codebook):
    batch = z.shape[0]
    idx, flat, com, cod, ent, perp = _search(z, codebook)
    table = codebook.reshape(G * K, D)
    gather_k = _make_sc_gather(batch * G, 512)
    q = gather_k(table, flat.reshape(-1))
    q = q.reshape(batch, G * D)
    return (q, idx, com.reshape(()), cod.reshape(()), ent.reshape(()),
            perp.reshape(()))


# speedup vs baseline: 2.0038x; 2.0038x over previous
import functools

import jax
import jax.numpy as jnp
from jax import lax
from jax.experimental import pallas as pl
from jax.experimental.pallas import tpu as pltpu
from jax.experimental.pallas import tpu_sc as plsc

G = 16
K = 1024
D = 64
TILE_B = 512


def _search_kernel(z_ref, cb_ref, idx_ref, flat_ref, com_ref, cod_ref,
                   ent_ref, perp_ref, csq_ref, loss_ref, hist_ref, *,
                   batch):
    i = pl.program_id(0)
    n_steps = pl.num_programs(0)

    @pl.when(i == 0)
    def _init():
        loss_ref[:, :] = jnp.zeros((1, 1), jnp.float32)
        hist_ref[:, :] = jnp.zeros((1, K), jnp.float32)
        for g in range(G):
            cb = cb_ref[g]
            csq_ref[g, :, :] = jnp.sum(cb * cb, axis=1)[None, :]

    z_all = z_ref[:, :]
    zneg = z_all * -2.0
    loss_part = jnp.zeros((1, 1), jnp.float32)
    hist_part = jnp.zeros((1, K), jnp.float32)
    ones_row = jnp.ones((1, TILE_B), jnp.float32)
    idx_cols = []
    flat_cols = []
    for g in range(G):
        zg = z_all[:, g * D:(g + 1) * D]
        cross2 = lax.dot_general(
            zneg[:, g * D:(g + 1) * D], cb_ref[g],
            (((1,), (1,)), ((), ())),
            preferred_element_type=jnp.float32)
        z_sq = jnp.sum(zg * zg, axis=1, keepdims=True)
        dist = (z_sq + cross2) + csq_ref[g]
        mind = jnp.min(dist, axis=1, keepdims=True)
        mask = dist <= mind
        iota = lax.broadcasted_iota(jnp.int32, (TILE_B, K), 1)
        idx = jnp.min(jnp.where(mask, iota, K), axis=1)
        idx = idx.astype(jnp.int32)
        maskf = jnp.where(mask, 1.0, 0.0).astype(jnp.float32)
        hist_part = hist_part + jnp.dot(
            ones_row, maskf, preferred_element_type=jnp.float32)
        loss_part = loss_part + jnp.sum(mind).reshape(1, 1)
        idx_cols.append(idx[:, None])
        flat_cols.append(idx[:, None] + g * K)

    idx_ref[:, :] = jnp.concatenate(idx_cols, axis=1)
    flat_ref[:, :] = jnp.concatenate(flat_cols, axis=1)
    hist_ref[:, :] += hist_part
    loss_ref[:, :] += loss_part

    @pl.when(i == n_steps - 1)
    def _finalize():
        loss = loss_ref[:, :] * (1.0 / jnp.float32(batch * G * D))
        com_ref[:, :] = loss
        cod_ref[:, :] = loss
        usage = hist_ref[:, :] * (1.0 / jnp.float32(batch * G))
        ent = -jnp.sum(usage * jnp.log(usage + 1e-10)).reshape(1, 1)
        ent_ref[:, :] = ent
        perp_ref[:, :] = jnp.exp(ent)


def _search(z, codebook):
    batch = z.shape[0]
    n_tiles = batch // TILE_B
    out_shapes = (
        jax.ShapeDtypeStruct((batch, G), jnp.int32),
        jax.ShapeDtypeStruct((batch, G), jnp.int32),
        jax.ShapeDtypeStruct((1, 1), jnp.float32),
        jax.ShapeDtypeStruct((1, 1), jnp.float32),
        jax.ShapeDtypeStruct((1, 1), jnp.float32),
        jax.ShapeDtypeStruct((1, 1), jnp.float32),
    )
    in_specs = [
        pl.BlockSpec((TILE_B, G * D), lambda i: (i, 0)),
        pl.BlockSpec((G, K, D), lambda i: (0, 0, 0)),
    ]
    out_specs = (
        pl.BlockSpec((TILE_B, G), lambda i: (i, 0)),
        pl.BlockSpec((TILE_B, G), lambda i: (i, 0)),
        pl.BlockSpec((1, 1), lambda i: (0, 0)),
        pl.BlockSpec((1, 1), lambda i: (0, 0)),
        pl.BlockSpec((1, 1), lambda i: (0, 0)),
        pl.BlockSpec((1, 1), lambda i: (0, 0)),
    )
    scratch_shapes = [
        pltpu.VMEM((G, 1, K), jnp.float32),
        pltpu.VMEM((1, 1), jnp.float32),
        pltpu.VMEM((1, K), jnp.float32),
    ]
    return pl.pallas_call(
        functools.partial(_search_kernel, batch=batch),
        grid=(n_tiles,),
        in_specs=in_specs,
        out_specs=out_specs,
        out_shape=out_shapes,
        scratch_shapes=scratch_shapes,
    )(z, codebook)


def _make_sc_gather(n_rows, chunk):
    info = plsc.get_sparse_core_info()
    nw = info.num_cores * info.num_subcores
    rows_per_w = n_rows // nw
    n_chunks = rows_per_w // chunk
    mesh = plsc.VectorSubcoreMesh(core_axis_name="c", subcore_axis_name="s")

    @functools.partial(
        pl.kernel, mesh=mesh,
        compiler_params=pltpu.CompilerParams(use_tc_tiling_on_sc=False),
        out_type=jax.ShapeDtypeStruct((n_rows, D), jnp.float32),
        scratch_types=[
            pltpu.VMEM((rows_per_w,), jnp.int32),
            pltpu.VMEM((chunk, D), jnp.float32),
            pltpu.VMEM((chunk, D), jnp.float32),
            pltpu.SemaphoreType.DMA,
            pltpu.SemaphoreType.DMA,
        ],
    )
    def gather_k(table_hbm, flat_hbm, out_hbm, idx_v, rows_a, rows_b,
                 sem_g, sem_s):
        wid = lax.axis_index("s") * info.num_cores + lax.axis_index("c")
        base = wid * rows_per_w
        pltpu.sync_copy(flat_hbm.at[pl.ds(base, rows_per_w)], idx_v)
        bufs = (rows_a, rows_b)
        prev_store = None
        for c in range(n_chunks):
            buf = bufs[c % 2]
            hg = pltpu.async_copy(
                table_hbm.at[idx_v.at[pl.ds(c * chunk, chunk)]], buf,
                sem_g)
            hg.wait()
            if prev_store is not None:
                prev_store.wait()
            prev_store = pltpu.async_copy(
                buf, out_hbm.at[pl.ds(base + c * chunk, chunk)], sem_s)
        prev_store.wait()

    return gather_k


def kernel(z, codebook):
    batch = z.shape[0]
    idx, flat, com, cod, ent, perp = _search(z, codebook)
    table = codebook.reshape(G * K, D)
    gather_k = _make_sc_gather(batch * G, 512)
    q = gather_k(table, flat.reshape(-1))
    q = q.reshape(batch, G * D)
    return (q, idx, com.reshape(()), cod.reshape(()), ent.reshape(()),
            perp.reshape(()))
